# baseline (speedup 1.0000x reference)
import functools

import jax
import jax.numpy as jnp
from jax import lax
from jax.experimental import pallas as pl
from jax.experimental.pallas import tpu as pltpu
from jax.experimental.pallas import tpu_sc as plsc

P_NUM, DR_NUM, DI_NUM = 10000, 8000, 5000
HID = 128
P_PAD, DR_PAD, DI_PAD = 10240, 8192, 5120

NCORE, NSUB = 2, 16
NW = NCORE * NSUB
CHUNK = 96
EBLK = NW * CHUNK * 4
EPS = 1e-16


def _ceil_to(x, m):
    return (x + m - 1) // m * m


@functools.lru_cache(maxsize=None)
def _edge_kernel(e_pad, n_src_pad, n_out_pad):
    ept = e_pad // NW
    nchunk = ept // CHUNK
    rpt = n_out_pad // NSUB
    nz = rpt // 64
    assert ept % CHUNK == 0 and rpt % 64 == 0 and nchunk % 4 == 0

    mesh = plsc.VectorSubcoreMesh(core_axis_name="c", subcore_axis_name="s",
                                  num_cores=NCORE, num_subcores=NSUB)

    idx_t = pltpu.VMEM((CHUNK,), jnp.int32)

    @functools.partial(
        pl.kernel, mesh=mesh,
        compiler_params=pltpu.CompilerParams(needs_layout_passes=False),
        out_type=(jax.ShapeDtypeStruct((NCORE, n_out_pad, HID), jnp.float32),
                  jax.ShapeDtypeStruct((NCORE, n_out_pad), jnp.float32)),
        scratch_types=[
            pltpu.VMEM((n_src_pad,), jnp.float32),
            pltpu.VMEM((n_out_pad,), jnp.float32),
            idx_t, idx_t, idx_t, idx_t,
            idx_t, idx_t, idx_t, idx_t,
            pltpu.VMEM((CHUNK + 16,), jnp.float32),
            pltpu.VMEM((CHUNK + 16,), jnp.float32),
            pltpu.VMEM((CHUNK, HID), jnp.float32),
            pltpu.VMEM((CHUNK, HID), jnp.float32),
            pltpu.VMEM_SHARED((n_out_pad, HID), jnp.float32),
            pltpu.VMEM_SHARED((n_out_pad,), jnp.float32),
            pltpu.SemaphoreType.DMA, pltpu.SemaphoreType.DMA,
            pltpu.SemaphoreType.DMA, pltpu.SemaphoreType.DMA,
            pltpu.SemaphoreType.DMA, pltpu.SemaphoreType.DMA,
            pltpu.SemaphoreType.DMA, pltpu.SemaphoreType.DMA,
        ])
    def k(src_hbm, seg_hbm, p_hbm, q_hbm, msg_hbm, num_hbm, d_hbm,
          p_l, q_l, s0, s1, s2, s3, g0, g1, g2, g3,
          e_b0, e_b1, rows_b0, rows_b1, num_sh, d_sh,
          gsem0, gsem1, ssem0, ssem1, isem0, isem1, isem2, isem3):
        cid = lax.axis_index("c")
        sid = lax.axis_index("s")
        wid = cid * NSUB + sid

        src_r = (s0, s1, s2, s3)
        seg_r = (g0, g1, g2, g3)
        isems = (isem0, isem1, isem2, isem3)
        rows_bufs = (rows_b0, rows_b1)
        e_bufs = (e_b0, e_b1)
        gsems = (gsem0, gsem1)
        ssems = (ssem0, ssem1)
        rows_b = rows_b0
        e_b = e_b0

        base = pl.multiple_of(wid * ept, CHUNK)
        pltpu.sync_copy(p_hbm, p_l)
        pltpu.sync_copy(q_hbm, q_l)

        zz = jnp.zeros((16,), jnp.float32)

        def zrow(r, _):
            for kk in range(HID // 16):
                rows_b[r, pl.ds(kk * 16, 16)] = zz
            return 0
        lax.fori_loop(0, CHUNK, zrow, 0)
        for kk in range(CHUNK // 16 + 1):
            e_b[pl.ds(kk * 16, 16)] = zz

        def zshared(j, _):
            r0 = sid * rpt + j * 64
            pltpu.sync_copy(rows_b.at[pl.ds(0, 64)], num_sh.at[pl.ds(r0, 64)])
            pltpu.sync_copy(e_b.at[pl.ds(0, 64)], d_sh.at[pl.ds(r0, 64)])
            return 0
        lax.fori_loop(0, nz, zshared, 0)
        plsc.subcore_barrier()

        def issue_idx(c, slot):
            off = pl.multiple_of(base + c * CHUNK, CHUNK)
            pltpu.async_copy(src_hbm.at[pl.ds(off, CHUNK)], src_r[slot],
                             isems[slot])
            pltpu.async_copy(seg_hbm.at[pl.ds(off, CHUNK)], seg_r[slot],
                             isems[slot])

        def wait_idx(slot):
            pltpu.make_async_copy(src_hbm.at[pl.ds(0, CHUNK)], src_r[slot],
                                  isems[slot]).wait()
            pltpu.make_async_copy(seg_hbm.at[pl.ds(0, CHUNK)], seg_r[slot],
                                  isems[slot]).wait()

        def issue_gather(slot, rb):
            pltpu.async_copy(msg_hbm.at[src_r[slot]], rows_bufs[rb],
                             gsems[rb])

        def wait_scatter(slot, rb):
            pltpu.make_async_copy(rows_bufs[rb], num_sh.at[seg_r[slot]],
                                  ssems[rb]).wait()
            pltpu.make_async_copy(e_bufs[rb].at[pl.ds(16, CHUNK)],
                                  d_sh.at[seg_r[slot]], ssems[rb]).wait()

        def compute(slot, rb):
            sb, gb = src_r[slot], seg_r[slot]
            rbuf, eb = rows_bufs[rb], e_bufs[rb]
            for i in range(CHUNK // 16):
                sv = sb[pl.ds(i * 16, 16)]
                gv = gb[pl.ds(i * 16, 16)]
                a = plsc.load_gather(p_l, [sv]) + plsc.load_gather(q_l, [gv])
                a = jnp.where(a > 0, a, 0.2 * a)
                eb[pl.ds(16 + i * 16, 16)] = jnp.exp(a)
            pltpu.make_async_copy(msg_hbm.at[sb], rbuf, gsems[rb]).wait()
            for r in range(CHUNK):
                s = plsc.load_gather(
                    eb, [jnp.full((16,), 16 + r, jnp.int32)])
                for kk in range(HID // 16):
                    rbuf[r, pl.ds(kk * 16, 16)] = (
                        rbuf[r, pl.ds(kk * 16, 16)] * s)

        def issue_scatter(slot, rb):
            gb = seg_r[slot]
            pltpu.async_copy(rows_bufs[rb], num_sh.at[gb], ssems[rb],
                             add=True)
            pltpu.async_copy(e_bufs[rb].at[pl.ds(16, CHUNK)], d_sh.at[gb],
                             ssems[rb], add=True)

        issue_idx(0, 0)
        issue_idx(1, 1)
        wait_idx(0)
        issue_gather(0, 0)

        def body(ci4, _):
            for u in range(4):
                c = ci4 * 4 + u
                nslot = (u + 2) % 4

                @pl.when(c + 2 < nchunk)
                def _():
                    issue_idx(c + 2, nslot)
                compute(u, u % 2)

                @pl.when(c >= 1)
                def _():
                    wait_scatter((u + 3) % 4, (u + 1) % 2)

                @pl.when(c + 1 < nchunk)
                def _():
                    wait_idx((u + 1) % 4)
                    issue_gather((u + 1) % 4, (u + 1) % 2)
                issue_scatter(u, u % 2)
            return 0
        lax.fori_loop(0, nchunk // 4, body, 0)
        wait_scatter((nchunk - 1) % 4, (nchunk - 1) % 2)
        plsc.subcore_barrier()

        def export(j, _):
            r0 = sid * rpt + j * 64
            pltpu.sync_copy(num_sh.at[pl.ds(r0, 64)],
                            num_hbm.at[cid].at[pl.ds(r0, 64)])
            pltpu.sync_copy(d_sh.at[pl.ds(r0, 64)], e_b.at[pl.ds(0, 64)])
            pltpu.sync_copy(e_b.at[pl.ds(0, 64)],
                            d_hbm.at[cid].at[pl.ds(r0, 64)])
            return 0
        lax.fori_loop(0, nz, export, 0)

    return k


_BN = 512


def _full(shape):
    return pl.BlockSpec(shape, lambda i: (0,) * len(shape))


def _rows(bn, w):
    return pl.BlockSpec((bn, w), lambda i: (i, 0))


def _mm_proj(x, w, a):
    n = x.shape[0]

    def body(x_ref, w_ref, a_ref, h_ref, s_ref):
        h = jnp.dot(x_ref[...], w_ref[...],
                    preferred_element_type=jnp.float32)
        h_ref[...] = h
        s_ref[...] = jnp.dot(h, a_ref[...],
                             preferred_element_type=jnp.float32)

    return pl.pallas_call(
        body, grid=(n // _BN,),
        in_specs=[_rows(_BN, x.shape[1]), _full(w.shape), _full(a.shape)],
        out_specs=[_rows(_BN, HID), _rows(_BN, 1)],
        out_shape=[jax.ShapeDtypeStruct((n, HID), jnp.float32),
                   jax.ShapeDtypeStruct((n, 1), jnp.float32)])(x, w, a)


def _matvec(x, u):
    n = x.shape[0]

    def body(x_ref, u_ref, s_ref):
        s_ref[...] = jnp.dot(x_ref[...], u_ref[...],
                             preferred_element_type=jnp.float32)

    return pl.pallas_call(
        body, grid=(n // _BN,),
        in_specs=[_rows(_BN, x.shape[1]), _full(u.shape)],
        out_specs=_rows(_BN, 1),
        out_shape=jax.ShapeDtypeStruct((n, 1), jnp.float32))(x, u)


def _epi(num, den, b):
    n = num.shape[1]
    n0, n1 = num[0], num[1]
    d0, d1 = den[0].reshape(n, 1), den[1].reshape(n, 1)
    b2 = b.reshape(1, HID)

    def body(n0_ref, n1_ref, d0_ref, d1_ref, b_ref, o_ref):
        d = d0_ref[...] + d1_ref[...] + EPS
        o_ref[...] = jnp.maximum(
            (n0_ref[...] + n1_ref[...]) / d + b_ref[...], 0.0)

    return pl.pallas_call(
        body, grid=(n // _BN,),
        in_specs=[_rows(_BN, HID), _rows(_BN, HID),
                  _rows(_BN, 1), _rows(_BN, 1), _full((1, HID))],
        out_specs=_rows(_BN, HID),
        out_shape=jax.ShapeDtypeStruct((n, HID), jnp.float32))(
            n0, n1, d0, d1, b2)


def _stage_ppi_proj(num_dp, d_dp, b_dp, num_xp, d_xp, b_xp,
                    w_dr, w_di, a_src, a_dst):
    n = num_dp.shape[1]
    args = (num_dp[0], num_dp[1], d_dp[0].reshape(n, 1), d_dp[1].reshape(n, 1),
            b_dp.reshape(1, HID),
            num_xp[0], num_xp[1], d_xp[0].reshape(n, 1), d_xp[1].reshape(n, 1),
            b_xp.reshape(1, HID),
            w_dr, w_di, a_src.reshape(HID, 1), a_dst.reshape(HID, 1))

    def body(ndp0, ndp1, ddp0, ddp1, bdp, nxp0, nxp1, dxp0, dxp1, bxp,
             wdr, wdi, asrc, adst, h_ref, ss_ref, sd_ref):
        pdr = jnp.maximum((ndp0[...] + ndp1[...])
                          / (ddp0[...] + ddp1[...] + EPS) + bdp[...], 0.0)
        pdi = jnp.maximum((nxp0[...] + nxp1[...])
                          / (dxp0[...] + dxp1[...] + EPS) + bxp[...], 0.0)
        h = (jnp.dot(pdr, wdr[...], preferred_element_type=jnp.float32)
             + jnp.dot(pdi, wdi[...], preferred_element_type=jnp.float32))
        h_ref[...] = h
        ss_ref[...] = jnp.dot(h, asrc[...], preferred_element_type=jnp.float32)
        sd_ref[...] = jnp.dot(h, adst[...], preferred_element_type=jnp.float32)

    return pl.pallas_call(
        body, grid=(n // _BN,),
        in_specs=[_rows(_BN, HID), _rows(_BN, HID), _rows(_BN, 1),
                  _rows(_BN, 1), _full((1, HID)),
                  _rows(_BN, HID), _rows(_BN, HID), _rows(_BN, 1),
                  _rows(_BN, 1), _full((1, HID)),
                  _full((HID, HID)), _full((HID, HID)),
                  _full((HID, 1)), _full((HID, 1))],
        out_specs=[_rows(_BN, HID), _rows(_BN, 1), _rows(_BN, 1)],
        out_shape=[jax.ShapeDtypeStruct((n, HID), jnp.float32),
                   jax.ShapeDtypeStruct((n, 1), jnp.float32),
                   jax.ShapeDtypeStruct((n, 1), jnp.float32)])(*args)


def _stage_phid_proj(num, den, b, pd_wb, pd_ab, px_wb, px_ab):
    n = num.shape[1]
    args = (num[0], num[1], den[0].reshape(n, 1), den[1].reshape(n, 1),
            b.reshape(1, HID), pd_wb, pd_ab.reshape(HID, 1),
            px_wb, px_ab.reshape(HID, 1))

    def body(n0, n1, d0, d1, bb, wpd, apd, wpx, apx,
             ph_ref, hpd_ref, spd_ref, hpx_ref, spx_ref):
        ph = jnp.maximum((n0[...] + n1[...])
                         / (d0[...] + d1[...] + EPS) + bb[...], 0.0)
        ph_ref[...] = ph
        hpd = jnp.dot(ph, wpd[...], preferred_element_type=jnp.float32)
        hpd_ref[...] = hpd
        spd_ref[...] = jnp.dot(hpd, apd[...],
                               preferred_element_type=jnp.float32)
        hpx = jnp.dot(ph, wpx[...], preferred_element_type=jnp.float32)
        hpx_ref[...] = hpx
        spx_ref[...] = jnp.dot(hpx, apx[...],
                               preferred_element_type=jnp.float32)

    return pl.pallas_call(
        body, grid=(n // _BN,),
        in_specs=[_rows(_BN, HID), _rows(_BN, HID), _rows(_BN, 1),
                  _rows(_BN, 1), _full((1, HID)),
                  _full((HID, HID)), _full((HID, 1)),
                  _full((HID, HID)), _full((HID, 1))],
        out_specs=[_rows(_BN, HID)] + [_rows(_BN, HID), _rows(_BN, 1)] * 2,
        out_shape=[jax.ShapeDtypeStruct((n, HID), jnp.float32),
                   jax.ShapeDtypeStruct((n, HID), jnp.float32),
                   jax.ShapeDtypeStruct((n, 1), jnp.float32),
                   jax.ShapeDtypeStruct((n, HID), jnp.float32),
                   jax.ShapeDtypeStruct((n, 1), jnp.float32)])(*args)


def _pad_edges(src, seg, e_pad, dummy_seg):
    e = src.shape[0]
    pad = e_pad - e
    src_p = jnp.concatenate(
        [src.astype(jnp.int32), jnp.zeros((pad,), jnp.int32)])
    seg_p = jnp.concatenate(
        [seg.astype(jnp.int32), jnp.full((pad,), dummy_seg, jnp.int32)])
    return src_p, seg_p


def _pad_rows(x, n_pad):
    return jnp.pad(x, ((0, n_pad - x.shape[0]), (0, 0)))


def kernel(ppi_edge_index, drug_protein_edge_index, disease_protein_edge_index,
           drug_feature, disease_feature, protein_feature,
           dp_Wa, dp_Wb, dp_aa, dp_ab, dp_b,
           xp_Wa, xp_Wb, xp_aa, xp_ab, xp_b,
           ppi_W, ppi_as, ppi_ad, ppi_b,
           pd_Wa, pd_Wb, pd_aa, pd_ab, pd_b,
           px_Wa, px_Wb, px_aa, px_ab, px_b):
    e_dp = drug_protein_edge_index.shape[1]
    e_xp = disease_protein_edge_index.shape[1]
    e_ppi = ppi_edge_index.shape[1] + P_NUM

    e_dp_pad = _ceil_to(e_dp, EBLK)
    e_xp_pad = _ceil_to(e_xp, EBLK)
    e_ppi_pad = _ceil_to(e_ppi, EBLK)

    drug_p = _pad_rows(drug_feature, DR_PAD)
    dis_p = _pad_rows(disease_feature, DI_PAD)
    prot_p = _pad_rows(protein_feature, P_PAD)

    ha_dp, sa_dp = _mm_proj(drug_p, dp_Wa, dp_aa.reshape(HID, 1))
    sb_dp = _matvec(prot_p, (dp_Wb @ dp_ab).reshape(HID, 1))
    ha_xp, sa_xp = _mm_proj(dis_p, xp_Wa, xp_aa.reshape(HID, 1))
    sb_xp = _matvec(prot_p, (xp_Wb @ xp_ab).reshape(HID, 1))

    dp_src, dp_seg = _pad_edges(drug_protein_edge_index[0],
                                drug_protein_edge_index[1], e_dp_pad, P_NUM)
    xp_src, xp_seg = _pad_edges(disease_protein_edge_index[0],
                                disease_protein_edge_index[1], e_xp_pad, P_NUM)

    num_dp, d_dp = _edge_kernel(e_dp_pad, DR_PAD, P_PAD)(
        dp_src, dp_seg, sa_dp.reshape(-1), sb_dp.reshape(-1), ha_dp)
    num_xp, d_xp = _edge_kernel(e_xp_pad, DI_PAD, P_PAD)(
        xp_src, xp_seg, sa_xp.reshape(-1), sb_xp.reshape(-1), ha_xp)

    h_ppi, s_src, s_dst = _stage_ppi_proj(
        num_dp, d_dp, dp_b, num_xp, d_xp, xp_b,
        ppi_W[:HID], ppi_W[HID:], ppi_as, ppi_ad)

    loops = jnp.arange(P_NUM, dtype=jnp.int32)
    ppi_src, ppi_seg = _pad_edges(
        jnp.concatenate([ppi_edge_index[0].astype(jnp.int32), loops]),
        jnp.concatenate([ppi_edge_index[1].astype(jnp.int32), loops]),
        e_ppi_pad, P_NUM)

    num_ppi, d_ppi = _edge_kernel(e_ppi_pad, P_PAD, P_PAD)(
        ppi_src, ppi_seg, s_src.reshape(-1), s_dst.reshape(-1), h_ppi)

    p_hid_pad, hb_pd, sb_pd, hb_px, sb_px = _stage_phid_proj(
        num_ppi, d_ppi, ppi_b, pd_Wb, pd_ab, px_Wb, px_ab)

    sa_pd = _matvec(drug_p, (pd_Wa @ pd_aa).reshape(HID, 1))
    sa_px = _matvec(dis_p, (px_Wa @ px_aa).reshape(HID, 1))

    pd_src, pd_seg = _pad_edges(drug_protein_edge_index[1],
                                drug_protein_edge_index[0], e_dp_pad, DR_NUM)
    px_src, px_seg = _pad_edges(disease_protein_edge_index[1],
                                disease_protein_edge_index[0], e_xp_pad,
                                DI_NUM)

    num_pd, d_pd = _edge_kernel(e_dp_pad, P_PAD, DR_PAD)(
        pd_src, pd_seg, sb_pd.reshape(-1), sa_pd.reshape(-1), hb_pd)
    num_px, d_px = _edge_kernel(e_xp_pad, P_PAD, DI_PAD)(
        px_src, px_seg, sb_px.reshape(-1), sa_px.reshape(-1), hb_px)

    drug_out = _epi(num_pd, d_pd, pd_b)[:DR_NUM]
    disease_out = _epi(num_px, d_px, px_b)[:DI_NUM]
    return (drug_out, disease_out, p_hid_pad[:P_NUM])


# speedup vs baseline: 1.7346x; 1.7346x over previous
import functools

import jax
import jax.numpy as jnp
from jax import lax
from jax.experimental import pallas as pl
from jax.experimental.pallas import tpu as pltpu
from jax.experimental.pallas import tpu_sc as plsc

P_NUM, DR_NUM, DI_NUM = 10000, 8000, 5000
HID = 128
P_PAD, DR_PAD, DI_PAD = 10240, 8192, 5120

NCORE, NSUB = 2, 16
NW = NCORE * NSUB
CHUNK = 64
EBLK = NW * CHUNK * 3
EPS = 1e-16


def _ceil_to(x, m):
    return (x + m - 1) // m * m


@functools.lru_cache(maxsize=None)
def _edge_kernel(e_pad, n_src_pad, n_out_pad):
    ept = e_pad // NW
    nchunk = ept // CHUNK
    rpt = n_out_pad // NSUB
    nz = rpt // 64
    assert ept % CHUNK == 0 and rpt % 64 == 0 and nchunk % 3 == 0

    mesh = plsc.VectorSubcoreMesh(core_axis_name="c", subcore_axis_name="s",
                                  num_cores=NCORE, num_subcores=NSUB)

    idx_t = pltpu.VMEM((CHUNK,), jnp.int32)
    e_t = pltpu.VMEM((CHUNK + 16,), jnp.float32)
    rows_t = pltpu.VMEM((CHUNK, HID), jnp.float32)
    sem_t = pltpu.SemaphoreType.DMA

    @functools.partial(
        pl.kernel, mesh=mesh,
        compiler_params=pltpu.CompilerParams(needs_layout_passes=False),
        out_type=(jax.ShapeDtypeStruct((NCORE, n_out_pad, HID), jnp.float32),
                  jax.ShapeDtypeStruct((NCORE, n_out_pad), jnp.float32)),
        scratch_types=[
            pltpu.VMEM((n_src_pad,), jnp.float32),
            pltpu.VMEM((n_out_pad,), jnp.float32),
            idx_t, idx_t, idx_t,
            idx_t, idx_t, idx_t,
            idx_t, idx_t, idx_t,
            e_t, e_t, e_t,
            rows_t, rows_t, rows_t,
            pltpu.VMEM_SHARED((n_out_pad, HID), jnp.float32),
            pltpu.VMEM_SHARED((n_out_pad,), jnp.float32),
            sem_t, sem_t, sem_t,
            sem_t, sem_t, sem_t,
            sem_t, sem_t, sem_t,
        ])
    def k(src_hbm, seg_hbm, p_hbm, q_hbm, msg_hbm, num_hbm, d_hbm,
          p_l, q_l, s0, s1, s2, g0, g1, g2, w0, w1, w2, e_b0, e_b1, e_b2,
          rows_b0, rows_b1, rows_b2, num_sh, d_sh,
          gsem0, gsem1, gsem2, ssem0, ssem1, ssem2, isem0, isem1, isem2):
        cid = lax.axis_index("c")
        sid = lax.axis_index("s")
        wid = cid * NSUB + sid

        src_r = (s0, s1, s2)
        seg_r = (g0, g1, g2)
        sseg_r = (w0, w1, w2)
        isems = (isem0, isem1, isem2)
        rows_bufs = (rows_b0, rows_b1, rows_b2)
        e_bufs = (e_b0, e_b1, e_b2)
        gsems = (gsem0, gsem1, gsem2)
        ssems = (ssem0, ssem1, ssem2)
        rows_b = rows_b0
        e_b = e_b0

        base = pl.multiple_of(wid * ept, CHUNK)
        pltpu.sync_copy(p_hbm, p_l)
        pltpu.sync_copy(q_hbm, q_l)

        zz = jnp.zeros((16,), jnp.float32)

        def zrow(r, _):
            for kk in range(HID // 16):
                rows_b[r, pl.ds(kk * 16, 16)] = zz
            return 0
        lax.fori_loop(0, CHUNK, zrow, 0)
        for kk in range(CHUNK // 16 + 1):
            e_b[pl.ds(kk * 16, 16)] = zz

        def zshared(j, _):
            r0 = sid * rpt + j * 64
            pltpu.sync_copy(rows_b.at[pl.ds(0, 64)], num_sh.at[pl.ds(r0, 64)])
            pltpu.sync_copy(e_b.at[pl.ds(0, 64)], d_sh.at[pl.ds(r0, 64)])
            return 0
        lax.fori_loop(0, nz, zshared, 0)
        plsc.subcore_barrier()

        def issue_idx(c, slot):
            off = pl.multiple_of(base + c * CHUNK, CHUNK)
            pltpu.async_copy(src_hbm.at[pl.ds(off, CHUNK)], src_r[slot],
                             isems[slot])
            pltpu.async_copy(seg_hbm.at[pl.ds(off, CHUNK)], seg_r[slot],
                             isems[slot])

        def wait_idx(slot):
            pltpu.make_async_copy(src_hbm.at[pl.ds(0, CHUNK)], src_r[slot],
                                  isems[slot]).wait()
            pltpu.make_async_copy(seg_hbm.at[pl.ds(0, CHUNK)], seg_r[slot],
                                  isems[slot]).wait()

        def issue_gather(slot):
            pltpu.async_copy(msg_hbm.at[src_r[slot]], rows_bufs[slot],
                             gsems[slot])

        def wait_scatter(slot):
            pltpu.make_async_copy(rows_bufs[slot], num_sh.at[sseg_r[slot]],
                                  ssems[slot]).wait()
            pltpu.make_async_copy(e_bufs[slot].at[pl.ds(16, CHUNK)],
                                  d_sh.at[sseg_r[slot]], ssems[slot]).wait()

        def compute(slot):
            sb, gb = src_r[slot], seg_r[slot]
            rbuf, eb, wb = rows_bufs[slot], e_bufs[slot], sseg_r[slot]
            for i in range(CHUNK // 16):
                sv = sb[pl.ds(i * 16, 16)]
                gv = gb[pl.ds(i * 16, 16)]
                a = plsc.load_gather(p_l, [sv]) + plsc.load_gather(q_l, [gv])
                a = jnp.where(a > 0, a, 0.2 * a)
                eb[pl.ds(16 + i * 16, 16)] = jnp.exp(a)
                wb[pl.ds(i * 16, 16)] = gv
            pltpu.make_async_copy(msg_hbm.at[sb], rbuf, gsems[slot]).wait()
            for r in range(CHUNK):
                s = plsc.load_gather(
                    eb, [jnp.full((16,), 16 + r, jnp.int32)])
                for kk in range(HID // 16):
                    rbuf[r, pl.ds(kk * 16, 16)] = (
                        rbuf[r, pl.ds(kk * 16, 16)] * s)

        def issue_scatter(slot):
            wb = sseg_r[slot]
            pltpu.async_copy(rows_bufs[slot], num_sh.at[wb], ssems[slot],
                             add=True)
            pltpu.async_copy(e_bufs[slot].at[pl.ds(16, CHUNK)], d_sh.at[wb],
                             ssems[slot], add=True)

        issue_idx(0, 0)
        issue_idx(1, 1)
        wait_idx(0)
        issue_gather(0)

        def body(ci3, _):
            for u in range(3):
                c = ci3 * 3 + u
                nx1 = (u + 1) % 3
                nx2 = (u + 2) % 3

                @pl.when(c >= 2)
                def _():
                    wait_scatter(nx1)

                @pl.when(c + 1 < nchunk)
                def _():
                    wait_idx(nx1)
                    issue_gather(nx1)

                @pl.when(c + 2 < nchunk)
                def _():
                    issue_idx(c + 2, nx2)
                compute(u)
                issue_scatter(u)
            return 0
        lax.fori_loop(0, nchunk // 3, body, 0)
        wait_scatter(1)
        wait_scatter(2)
        plsc.subcore_barrier()

        def export(j, _):
            r0 = sid * rpt + j * 64
            pltpu.sync_copy(num_sh.at[pl.ds(r0, 64)],
                            num_hbm.at[cid].at[pl.ds(r0, 64)])
            pltpu.sync_copy(d_sh.at[pl.ds(r0, 64)], e_b.at[pl.ds(0, 64)])
            pltpu.sync_copy(e_b.at[pl.ds(0, 64)],
                            d_hbm.at[cid].at[pl.ds(r0, 64)])
            return 0
        lax.fori_loop(0, nz, export, 0)

    return k


_BN = 512


def _full(shape):
    return pl.BlockSpec(shape, lambda i: (0,) * len(shape))


def _rows(bn, w):
    return pl.BlockSpec((bn, w), lambda i: (i, 0))


def _mm_proj(x, w, a):
    n = x.shape[0]

    def body(x_ref, w_ref, a_ref, h_ref, s_ref):
        h = jnp.dot(x_ref[...], w_ref[...],
                    preferred_element_type=jnp.float32)
        h_ref[...] = h
        s_ref[...] = jnp.dot(h, a_ref[...],
                             preferred_element_type=jnp.float32)

    return pl.pallas_call(
        body, grid=(n // _BN,),
        in_specs=[_rows(_BN, x.shape[1]), _full(w.shape), _full(a.shape)],
        out_specs=[_rows(_BN, HID), _rows(_BN, 1)],
        out_shape=[jax.ShapeDtypeStruct((n, HID), jnp.float32),
                   jax.ShapeDtypeStruct((n, 1), jnp.float32)])(x, w, a)


def _matvec(x, u):
    n = x.shape[0]

    def body(x_ref, u_ref, s_ref):
        s_ref[...] = jnp.dot(x_ref[...], u_ref[...],
                             preferred_element_type=jnp.float32)

    return pl.pallas_call(
        body, grid=(n // _BN,),
        in_specs=[_rows(_BN, x.shape[1]), _full(u.shape)],
        out_specs=_rows(_BN, 1),
        out_shape=jax.ShapeDtypeStruct((n, 1), jnp.float32))(x, u)


def _epi(num, den, b):
    n = num.shape[1]
    n0, n1 = num[0], num[1]
    d0, d1 = den[0].reshape(n, 1), den[1].reshape(n, 1)
    b2 = b.reshape(1, HID)

    def body(n0_ref, n1_ref, d0_ref, d1_ref, b_ref, o_ref):
        d = d0_ref[...] + d1_ref[...] + EPS
        o_ref[...] = jnp.maximum(
            (n0_ref[...] + n1_ref[...]) / d + b_ref[...], 0.0)

    return pl.pallas_call(
        body, grid=(n // _BN,),
        in_specs=[_rows(_BN, HID), _rows(_BN, HID),
                  _rows(_BN, 1), _rows(_BN, 1), _full((1, HID))],
        out_specs=_rows(_BN, HID),
        out_shape=jax.ShapeDtypeStruct((n, HID), jnp.float32))(
            n0, n1, d0, d1, b2)


def _stage_ppi_proj(num_dp, d_dp, b_dp, num_xp, d_xp, b_xp,
                    w_dr, w_di, a_src, a_dst):
    n = num_dp.shape[1]
    args = (num_dp[0], num_dp[1], d_dp[0].reshape(n, 1), d_dp[1].reshape(n, 1),
            b_dp.reshape(1, HID),
            num_xp[0], num_xp[1], d_xp[0].reshape(n, 1), d_xp[1].reshape(n, 1),
            b_xp.reshape(1, HID),
            w_dr, w_di, a_src.reshape(HID, 1), a_dst.reshape(HID, 1))

    def body(ndp0, ndp1, ddp0, ddp1, bdp, nxp0, nxp1, dxp0, dxp1, bxp,
             wdr, wdi, asrc, adst, h_ref, ss_ref, sd_ref):
        pdr = jnp.maximum((ndp0[...] + ndp1[...])
                          / (ddp0[...] + ddp1[...] + EPS) + bdp[...], 0.0)
        pdi = jnp.maximum((nxp0[...] + nxp1[...])
                          / (dxp0[...] + dxp1[...] + EPS) + bxp[...], 0.0)
        h = (jnp.dot(pdr, wdr[...], preferred_element_type=jnp.float32)
             + jnp.dot(pdi, wdi[...], preferred_element_type=jnp.float32))
        h_ref[...] = h
        ss_ref[...] = jnp.dot(h, asrc[...], preferred_element_type=jnp.float32)
        sd_ref[...] = jnp.dot(h, adst[...], preferred_element_type=jnp.float32)

    return pl.pallas_call(
        body, grid=(n // _BN,),
        in_specs=[_rows(_BN, HID), _rows(_BN, HID), _rows(_BN, 1),
                  _rows(_BN, 1), _full((1, HID)),
                  _rows(_BN, HID), _rows(_BN, HID), _rows(_BN, 1),
                  _rows(_BN, 1), _full((1, HID)),
                  _full((HID, HID)), _full((HID, HID)),
                  _full((HID, 1)), _full((HID, 1))],
        out_specs=[_rows(_BN, HID), _rows(_BN, 1), _rows(_BN, 1)],
        out_shape=[jax.ShapeDtypeStruct((n, HID), jnp.float32),
                   jax.ShapeDtypeStruct((n, 1), jnp.float32),
                   jax.ShapeDtypeStruct((n, 1), jnp.float32)])(*args)


def _stage_phid_proj(num, den, b, pd_wb, pd_ab, px_wb, px_ab):
    n = num.shape[1]
    args = (num[0], num[1], den[0].reshape(n, 1), den[1].reshape(n, 1),
            b.reshape(1, HID), pd_wb, pd_ab.reshape(HID, 1),
            px_wb, px_ab.reshape(HID, 1))

    def body(n0, n1, d0, d1, bb, wpd, apd, wpx, apx,
             ph_ref, hpd_ref, spd_ref, hpx_ref, spx_ref):
        ph = jnp.maximum((n0[...] + n1[...])
                         / (d0[...] + d1[...] + EPS) + bb[...], 0.0)
        ph_ref[...] = ph
        hpd = jnp.dot(ph, wpd[...], preferred_element_type=jnp.float32)
        hpd_ref[...] = hpd
        spd_ref[...] = jnp.dot(hpd, apd[...],
                               preferred_element_type=jnp.float32)
        hpx = jnp.dot(ph, wpx[...], preferred_element_type=jnp.float32)
        hpx_ref[...] = hpx
        spx_ref[...] = jnp.dot(hpx, apx[...],
                               preferred_element_type=jnp.float32)

    return pl.pallas_call(
        body, grid=(n // _BN,),
        in_specs=[_rows(_BN, HID), _rows(_BN, HID), _rows(_BN, 1),
                  _rows(_BN, 1), _full((1, HID)),
                  _full((HID, HID)), _full((HID, 1)),
                  _full((HID, HID)), _full((HID, 1))],
        out_specs=[_rows(_BN, HID)] + [_rows(_BN, HID), _rows(_BN, 1)] * 2,
        out_shape=[jax.ShapeDtypeStruct((n, HID), jnp.float32),
                   jax.ShapeDtypeStruct((n, HID), jnp.float32),
                   jax.ShapeDtypeStruct((n, 1), jnp.float32),
                   jax.ShapeDtypeStruct((n, HID), jnp.float32),
                   jax.ShapeDtypeStruct((n, 1), jnp.float32)])(*args)


def _pad_edges(src, seg, e_pad, dummy_seg):
    e = src.shape[0]
    pad = e_pad - e
    src_p = jnp.concatenate(
        [src.astype(jnp.int32), jnp.zeros((pad,), jnp.int32)])
    seg_p = jnp.concatenate(
        [seg.astype(jnp.int32), jnp.full((pad,), dummy_seg, jnp.int32)])
    return src_p, seg_p


def _pad_rows(x, n_pad):
    return jnp.pad(x, ((0, n_pad - x.shape[0]), (0, 0)))


def kernel(ppi_edge_index, drug_protein_edge_index, disease_protein_edge_index,
           drug_feature, disease_feature, protein_feature,
           dp_Wa, dp_Wb, dp_aa, dp_ab, dp_b,
           xp_Wa, xp_Wb, xp_aa, xp_ab, xp_b,
           ppi_W, ppi_as, ppi_ad, ppi_b,
           pd_Wa, pd_Wb, pd_aa, pd_ab, pd_b,
           px_Wa, px_Wb, px_aa, px_ab, px_b):
    e_dp = drug_protein_edge_index.shape[1]
    e_xp = disease_protein_edge_index.shape[1]
    e_ppi = ppi_edge_index.shape[1] + P_NUM

    e_dp_pad = _ceil_to(e_dp, EBLK)
    e_xp_pad = _ceil_to(e_xp, EBLK)
    e_ppi_pad = _ceil_to(e_ppi, EBLK)

    drug_p = _pad_rows(drug_feature, DR_PAD)
    dis_p = _pad_rows(disease_feature, DI_PAD)
    prot_p = _pad_rows(protein_feature, P_PAD)

    ha_dp, sa_dp = _mm_proj(drug_p, dp_Wa, dp_aa.reshape(HID, 1))
    sb_dp = _matvec(prot_p, (dp_Wb @ dp_ab).reshape(HID, 1))
    ha_xp, sa_xp = _mm_proj(dis_p, xp_Wa, xp_aa.reshape(HID, 1))
    sb_xp = _matvec(prot_p, (xp_Wb @ xp_ab).reshape(HID, 1))

    dp_src, dp_seg = _pad_edges(drug_protein_edge_index[0],
                                drug_protein_edge_index[1], e_dp_pad, P_NUM)
    xp_src, xp_seg = _pad_edges(disease_protein_edge_index[0],
                                disease_protein_edge_index[1], e_xp_pad, P_NUM)

    num_dp, d_dp = _edge_kernel(e_dp_pad, DR_PAD, P_PAD)(
        dp_src, dp_seg, sa_dp.reshape(-1), sb_dp.reshape(-1), ha_dp)
    sa_xp_t, _ = lax.optimization_barrier((sa_xp.reshape(-1), d_dp))
    num_xp, d_xp = _edge_kernel(e_xp_pad, DI_PAD, P_PAD)(
        xp_src, xp_seg, sa_xp_t, sb_xp.reshape(-1), ha_xp)

    h_ppi, s_src, s_dst = _stage_ppi_proj(
        num_dp, d_dp, dp_b, num_xp, d_xp, xp_b,
        ppi_W[:HID], ppi_W[HID:], ppi_as, ppi_ad)

    loops = jnp.arange(P_NUM, dtype=jnp.int32)
    ppi_src, ppi_seg = _pad_edges(
        jnp.concatenate([ppi_edge_index[0].astype(jnp.int32), loops]),
        jnp.concatenate([ppi_edge_index[1].astype(jnp.int32), loops]),
        e_ppi_pad, P_NUM)

    num_ppi, d_ppi = _edge_kernel(e_ppi_pad, P_PAD, P_PAD)(
        ppi_src, ppi_seg, s_src.reshape(-1), s_dst.reshape(-1), h_ppi)

    p_hid_pad, hb_pd, sb_pd, hb_px, sb_px = _stage_phid_proj(
        num_ppi, d_ppi, ppi_b, pd_Wb, pd_ab, px_Wb, px_ab)

    sa_pd = _matvec(drug_p, (pd_Wa @ pd_aa).reshape(HID, 1))
    sa_px = _matvec(dis_p, (px_Wa @ px_aa).reshape(HID, 1))

    pd_src, pd_seg = _pad_edges(drug_protein_edge_index[1],
                                drug_protein_edge_index[0], e_dp_pad, DR_NUM)
    px_src, px_seg = _pad_edges(disease_protein_edge_index[1],
                                disease_protein_edge_index[0], e_xp_pad,
                                DI_NUM)

    num_pd, d_pd = _edge_kernel(e_dp_pad, P_PAD, DR_PAD)(
        pd_src, pd_seg, sb_pd.reshape(-1), sa_pd.reshape(-1), hb_pd)
    sb_px_t, _ = lax.optimization_barrier((sb_px.reshape(-1), d_pd))
    num_px, d_px = _edge_kernel(e_xp_pad, P_PAD, DI_PAD)(
        px_src, px_seg, sb_px_t, sa_px.reshape(-1), hb_px)

    drug_out = _epi(num_pd, d_pd, pd_b)[:DR_NUM]
    disease_out = _epi(num_px, d_px, px_b)[:DI_NUM]
    return (drug_out, disease_out, p_hid_pad[:P_NUM])


# speedup vs baseline: 1.7541x; 1.0113x over previous
import functools

import jax
import jax.numpy as jnp
from jax import lax
from jax.experimental import pallas as pl
from jax.experimental.pallas import tpu as pltpu
from jax.experimental.pallas import tpu_sc as plsc

P_NUM, DR_NUM, DI_NUM = 10000, 8000, 5000
HID = 128
P_PAD, DR_PAD, DI_PAD = 10240, 8192, 5120

NCORE, NSUB = 2, 16
NW = NCORE * NSUB
CHUNK = 64
EBLK = NW * CHUNK * 3
EPS = 1e-16


def _ceil_to(x, m):
    return (x + m - 1) // m * m


@functools.lru_cache(maxsize=None)
def _edge_kernel(e_pad, n_src_pad, n_out_pad):
    ept = e_pad // NW
    nchunk = ept // CHUNK
    rpt = n_out_pad // NSUB
    nz = rpt // 64
    assert ept % CHUNK == 0 and rpt % 64 == 0 and nchunk % 3 == 0

    mesh = plsc.VectorSubcoreMesh(core_axis_name="c", subcore_axis_name="s",
                                  num_cores=NCORE, num_subcores=NSUB)

    idx_t = pltpu.VMEM((CHUNK,), jnp.int32)
    e_t = pltpu.VMEM((CHUNK + 16,), jnp.float32)
    rows_t = pltpu.VMEM((CHUNK, HID), jnp.float32)
    sem_t = pltpu.SemaphoreType.DMA

    @functools.partial(
        pl.kernel, mesh=mesh,
        compiler_params=pltpu.CompilerParams(needs_layout_passes=False),
        out_type=(jax.ShapeDtypeStruct((NCORE, n_out_pad, HID), jnp.float32),
                  jax.ShapeDtypeStruct((NCORE, n_out_pad), jnp.float32)),
        scratch_types=[
            pltpu.VMEM((n_src_pad,), jnp.float32),
            pltpu.VMEM((n_out_pad,), jnp.float32),
            idx_t, idx_t, idx_t,
            idx_t, idx_t, idx_t,
            idx_t, idx_t, idx_t,
            e_t, e_t, e_t,
            rows_t, rows_t, rows_t,
            pltpu.VMEM_SHARED((n_out_pad, HID), jnp.float32),
            pltpu.VMEM_SHARED((n_out_pad,), jnp.float32),
            sem_t, sem_t, sem_t,
            sem_t, sem_t, sem_t,
            sem_t, sem_t, sem_t,
        ])
    def k(src_hbm, seg_hbm, p_hbm, q_hbm, msg_hbm, num_hbm, d_hbm,
          p_l, q_l, s0, s1, s2, g0, g1, g2, w0, w1, w2, e_b0, e_b1, e_b2,
          rows_b0, rows_b1, rows_b2, num_sh, d_sh,
          gsem0, gsem1, gsem2, ssem0, ssem1, ssem2, isem0, isem1, isem2):
        cid = lax.axis_index("c")
        sid = lax.axis_index("s")
        wid = cid * NSUB + sid

        src_r = (s0, s1, s2)
        seg_r = (g0, g1, g2)
        sseg_r = (w0, w1, w2)
        isems = (isem0, isem1, isem2)
        rows_bufs = (rows_b0, rows_b1, rows_b2)
        e_bufs = (e_b0, e_b1, e_b2)
        gsems = (gsem0, gsem1, gsem2)
        ssems = (ssem0, ssem1, ssem2)
        rows_b = rows_b0
        e_b = e_b0

        base = pl.multiple_of(wid * ept, CHUNK)
        pltpu.async_copy(p_hbm, p_l, isem0)
        pltpu.async_copy(q_hbm, q_l, isem1)

        zz = jnp.zeros((16,), jnp.float32)

        def zrow(r, _):
            for kk in range(HID // 16):
                rows_b[r, pl.ds(kk * 16, 16)] = zz
            return 0
        lax.fori_loop(0, CHUNK, zrow, 0)
        for kk in range(CHUNK // 16 + 1):
            e_b[pl.ds(kk * 16, 16)] = zz

        def zissue(j, _):
            r0 = sid * rpt + j * 64
            pltpu.async_copy(rows_b.at[pl.ds(0, 64)],
                             num_sh.at[pl.ds(r0, 64)], gsem0)
            pltpu.async_copy(e_b.at[pl.ds(0, 64)],
                             d_sh.at[pl.ds(r0, 64)], gsem1)
            return 0
        lax.fori_loop(0, nz, zissue, 0)

        def zwait(j, _):
            pltpu.make_async_copy(rows_b.at[pl.ds(0, 64)],
                                  num_sh.at[pl.ds(0, 64)], gsem0).wait()
            pltpu.make_async_copy(e_b.at[pl.ds(0, 64)],
                                  d_sh.at[pl.ds(0, 64)], gsem1).wait()
            return 0
        lax.fori_loop(0, nz, zwait, 0)
        pltpu.make_async_copy(p_hbm, p_l, isem0).wait()
        pltpu.make_async_copy(q_hbm, q_l, isem1).wait()
        plsc.subcore_barrier()

        def issue_idx(c, slot):
            off = pl.multiple_of(base + c * CHUNK, CHUNK)
            pltpu.async_copy(src_hbm.at[pl.ds(off, CHUNK)], src_r[slot],
                             isems[slot])
            pltpu.async_copy(seg_hbm.at[pl.ds(off, CHUNK)], seg_r[slot],
                             isems[slot])

        def wait_idx(slot):
            pltpu.make_async_copy(src_hbm.at[pl.ds(0, CHUNK)], src_r[slot],
                                  isems[slot]).wait()
            pltpu.make_async_copy(seg_hbm.at[pl.ds(0, CHUNK)], seg_r[slot],
                                  isems[slot]).wait()

        def issue_gather(slot):
            pltpu.async_copy(msg_hbm.at[src_r[slot]], rows_bufs[slot],
                             gsems[slot])

        def wait_scatter(slot):
            pltpu.make_async_copy(rows_bufs[slot], num_sh.at[sseg_r[slot]],
                                  ssems[slot]).wait()
            pltpu.make_async_copy(e_bufs[slot].at[pl.ds(16, CHUNK)],
                                  d_sh.at[sseg_r[slot]], ssems[slot]).wait()

        def compute(slot):
            sb, gb = src_r[slot], seg_r[slot]
            rbuf, eb, wb = rows_bufs[slot], e_bufs[slot], sseg_r[slot]
            for i in range(CHUNK // 16):
                sv = sb[pl.ds(i * 16, 16)]
                gv = gb[pl.ds(i * 16, 16)]
                a = plsc.load_gather(p_l, [sv]) + plsc.load_gather(q_l, [gv])
                a = jnp.where(a > 0, a, 0.2 * a)
                eb[pl.ds(16 + i * 16, 16)] = jnp.exp(a)
                wb[pl.ds(i * 16, 16)] = gv
            pltpu.make_async_copy(msg_hbm.at[sb], rbuf, gsems[slot]).wait()
            for r in range(CHUNK):
                s = plsc.load_gather(
                    eb, [jnp.full((16,), 16 + r, jnp.int32)])
                for kk in range(HID // 16):
                    rbuf[r, pl.ds(kk * 16, 16)] = (
                        rbuf[r, pl.ds(kk * 16, 16)] * s)

        def issue_scatter(slot):
            wb = sseg_r[slot]
            pltpu.async_copy(rows_bufs[slot], num_sh.at[wb], ssems[slot],
                             add=True)
            pltpu.async_copy(e_bufs[slot].at[pl.ds(16, CHUNK)], d_sh.at[wb],
                             ssems[slot], add=True)

        issue_idx(0, 0)
        issue_idx(1, 1)
        wait_idx(0)
        issue_gather(0)

        def body(ci3, _):
            for u in range(3):
                c = ci3 * 3 + u
                nx1 = (u + 1) % 3
                nx2 = (u + 2) % 3

                @pl.when(c >= 2)
                def _():
                    wait_scatter(nx1)

                @pl.when(c + 1 < nchunk)
                def _():
                    wait_idx(nx1)
                    issue_gather(nx1)

                @pl.when(c + 2 < nchunk)
                def _():
                    issue_idx(c + 2, nx2)
                compute(u)
                issue_scatter(u)
            return 0
        lax.fori_loop(0, nchunk // 3, body, 0)
        wait_scatter(1)
        wait_scatter(2)
        plsc.subcore_barrier()

        def export(j, _):
            r0 = sid * rpt + j * 64
            pltpu.sync_copy(num_sh.at[pl.ds(r0, 64)],
                            num_hbm.at[cid].at[pl.ds(r0, 64)])
            pltpu.sync_copy(d_sh.at[pl.ds(r0, 64)], e_b.at[pl.ds(0, 64)])
            pltpu.sync_copy(e_b.at[pl.ds(0, 64)],
                            d_hbm.at[cid].at[pl.ds(r0, 64)])
            return 0
        lax.fori_loop(0, nz, export, 0)

    return k


_BN = 512


def _full(shape):
    return pl.BlockSpec(shape, lambda i: (0,) * len(shape))


def _rows(bn, w):
    return pl.BlockSpec((bn, w), lambda i: (i, 0))


def _mm_proj(x, w, a):
    n = x.shape[0]

    def body(x_ref, w_ref, a_ref, h_ref, s_ref):
        h = jnp.dot(x_ref[...], w_ref[...],
                    preferred_element_type=jnp.float32)
        h_ref[...] = h
        s_ref[...] = jnp.dot(h, a_ref[...],
                             preferred_element_type=jnp.float32)

    return pl.pallas_call(
        body, grid=(n // _BN,),
        in_specs=[_rows(_BN, x.shape[1]), _full(w.shape), _full(a.shape)],
        out_specs=[_rows(_BN, HID), _rows(_BN, 1)],
        out_shape=[jax.ShapeDtypeStruct((n, HID), jnp.float32),
                   jax.ShapeDtypeStruct((n, 1), jnp.float32)])(x, w, a)


def _matvec(x, u):
    n = x.shape[0]

    def body(x_ref, u_ref, s_ref):
        s_ref[...] = jnp.dot(x_ref[...], u_ref[...],
                             preferred_element_type=jnp.float32)

    return pl.pallas_call(
        body, grid=(n // _BN,),
        in_specs=[_rows(_BN, x.shape[1]), _full(u.shape)],
        out_specs=_rows(_BN, 1),
        out_shape=jax.ShapeDtypeStruct((n, 1), jnp.float32))(x, u)


def _epi(num, den, b):
    n = num.shape[1]
    n0, n1 = num[0], num[1]
    d0, d1 = den[0].reshape(n, 1), den[1].reshape(n, 1)
    b2 = b.reshape(1, HID)

    def body(n0_ref, n1_ref, d0_ref, d1_ref, b_ref, o_ref):
        d = d0_ref[...] + d1_ref[...] + EPS
        o_ref[...] = jnp.maximum(
            (n0_ref[...] + n1_ref[...]) / d + b_ref[...], 0.0)

    return pl.pallas_call(
        body, grid=(n // _BN,),
        in_specs=[_rows(_BN, HID), _rows(_BN, HID),
                  _rows(_BN, 1), _rows(_BN, 1), _full((1, HID))],
        out_specs=_rows(_BN, HID),
        out_shape=jax.ShapeDtypeStruct((n, HID), jnp.float32))(
            n0, n1, d0, d1, b2)


def _stage_ppi_proj(num_dp, d_dp, b_dp, num_xp, d_xp, b_xp,
                    w_dr, w_di, a_src, a_dst):
    n = num_dp.shape[1]
    args = (num_dp[0], num_dp[1], d_dp[0].reshape(n, 1), d_dp[1].reshape(n, 1),
            b_dp.reshape(1, HID),
            num_xp[0], num_xp[1], d_xp[0].reshape(n, 1), d_xp[1].reshape(n, 1),
            b_xp.reshape(1, HID),
            w_dr, w_di, a_src.reshape(HID, 1), a_dst.reshape(HID, 1))

    def body(ndp0, ndp1, ddp0, ddp1, bdp, nxp0, nxp1, dxp0, dxp1, bxp,
             wdr, wdi, asrc, adst, h_ref, ss_ref, sd_ref):
        pdr = jnp.maximum((ndp0[...] + ndp1[...])
                          / (ddp0[...] + ddp1[...] + EPS) + bdp[...], 0.0)
        pdi = jnp.maximum((nxp0[...] + nxp1[...])
                          / (dxp0[...] + dxp1[...] + EPS) + bxp[...], 0.0)
        h = (jnp.dot(pdr, wdr[...], preferred_element_type=jnp.float32)
             + jnp.dot(pdi, wdi[...], preferred_element_type=jnp.float32))
        h_ref[...] = h
        ss_ref[...] = jnp.dot(h, asrc[...], preferred_element_type=jnp.float32)
        sd_ref[...] = jnp.dot(h, adst[...], preferred_element_type=jnp.float32)

    return pl.pallas_call(
        body, grid=(n // _BN,),
        in_specs=[_rows(_BN, HID), _rows(_BN, HID), _rows(_BN, 1),
                  _rows(_BN, 1), _full((1, HID)),
                  _rows(_BN, HID), _rows(_BN, HID), _rows(_BN, 1),
                  _rows(_BN, 1), _full((1, HID)),
                  _full((HID, HID)), _full((HID, HID)),
                  _full((HID, 1)), _full((HID, 1))],
        out_specs=[_rows(_BN, HID), _rows(_BN, 1), _rows(_BN, 1)],
        out_shape=[jax.ShapeDtypeStruct((n, HID), jnp.float32),
                   jax.ShapeDtypeStruct((n, 1), jnp.float32),
                   jax.ShapeDtypeStruct((n, 1), jnp.float32)])(*args)


def _stage_phid_proj(num, den, b, pd_wb, pd_ab, px_wb, px_ab):
    n = num.shape[1]
    args = (num[0], num[1], den[0].reshape(n, 1), den[1].reshape(n, 1),
            b.reshape(1, HID), pd_wb, pd_ab.reshape(HID, 1),
            px_wb, px_ab.reshape(HID, 1))

    def body(n0, n1, d0, d1, bb, wpd, apd, wpx, apx,
             ph_ref, hpd_ref, spd_ref, hpx_ref, spx_ref):
        ph = jnp.maximum((n0[...] + n1[...])
                         / (d0[...] + d1[...] + EPS) + bb[...], 0.0)
        ph_ref[...] = ph
        hpd = jnp.dot(ph, wpd[...], preferred_element_type=jnp.float32)
        hpd_ref[...] = hpd
        spd_ref[...] = jnp.dot(hpd, apd[...],
                               preferred_element_type=jnp.float32)
        hpx = jnp.dot(ph, wpx[...], preferred_element_type=jnp.float32)
        hpx_ref[...] = hpx
        spx_ref[...] = jnp.dot(hpx, apx[...],
                               preferred_element_type=jnp.float32)

    return pl.pallas_call(
        body, grid=(n // _BN,),
        in_specs=[_rows(_BN, HID), _rows(_BN, HID), _rows(_BN, 1),
                  _rows(_BN, 1), _full((1, HID)),
                  _full((HID, HID)), _full((HID, 1)),
                  _full((HID, HID)), _full((HID, 1))],
        out_specs=[_rows(_BN, HID)] + [_rows(_BN, HID), _rows(_BN, 1)] * 2,
        out_shape=[jax.ShapeDtypeStruct((n, HID), jnp.float32),
                   jax.ShapeDtypeStruct((n, HID), jnp.float32),
                   jax.ShapeDtypeStruct((n, 1), jnp.float32),
                   jax.ShapeDtypeStruct((n, HID), jnp.float32),
                   jax.ShapeDtypeStruct((n, 1), jnp.float32)])(*args)


def _pad_edges(src, seg, e_pad, dummy_seg):
    e = src.shape[0]
    pad = e_pad - e
    src_p = jnp.concatenate(
        [src.astype(jnp.int32), jnp.zeros((pad,), jnp.int32)])
    seg_p = jnp.concatenate(
        [seg.astype(jnp.int32), jnp.full((pad,), dummy_seg, jnp.int32)])
    return src_p, seg_p


def _pad_rows(x, n_pad):
    return jnp.pad(x, ((0, n_pad - x.shape[0]), (0, 0)))


def kernel(ppi_edge_index, drug_protein_edge_index, disease_protein_edge_index,
           drug_feature, disease_feature, protein_feature,
           dp_Wa, dp_Wb, dp_aa, dp_ab, dp_b,
           xp_Wa, xp_Wb, xp_aa, xp_ab, xp_b,
           ppi_W, ppi_as, ppi_ad, ppi_b,
           pd_Wa, pd_Wb, pd_aa, pd_ab, pd_b,
           px_Wa, px_Wb, px_aa, px_ab, px_b):
    e_dp = drug_protein_edge_index.shape[1]
    e_xp = disease_protein_edge_index.shape[1]
    e_ppi = ppi_edge_index.shape[1] + P_NUM

    e_dp_pad = _ceil_to(e_dp, EBLK)
    e_xp_pad = _ceil_to(e_xp, EBLK)
    e_ppi_pad = _ceil_to(e_ppi, EBLK)

    drug_p = _pad_rows(drug_feature, DR_PAD)
    dis_p = _pad_rows(disease_feature, DI_PAD)
    prot_p = _pad_rows(protein_feature, P_PAD)

    ha_dp, sa_dp = _mm_proj(drug_p, dp_Wa, dp_aa.reshape(HID, 1))
    sb_dp = _matvec(prot_p, (dp_Wb @ dp_ab).reshape(HID, 1))
    ha_xp, sa_xp = _mm_proj(dis_p, xp_Wa, xp_aa.reshape(HID, 1))
    sb_xp = _matvec(prot_p, (xp_Wb @ xp_ab).reshape(HID, 1))

    dp_src, dp_seg = _pad_edges(drug_protein_edge_index[0],
                                drug_protein_edge_index[1], e_dp_pad, P_NUM)
    xp_src, xp_seg = _pad_edges(disease_protein_edge_index[0],
                                disease_protein_edge_index[1], e_xp_pad, P_NUM)

    num_dp, d_dp = _edge_kernel(e_dp_pad, DR_PAD, P_PAD)(
        dp_src, dp_seg, sa_dp.reshape(-1), sb_dp.reshape(-1), ha_dp)
    sa_xp_t, _ = lax.optimization_barrier((sa_xp.reshape(-1), d_dp))
    num_xp, d_xp = _edge_kernel(e_xp_pad, DI_PAD, P_PAD)(
        xp_src, xp_seg, sa_xp_t, sb_xp.reshape(-1), ha_xp)

    h_ppi, s_src, s_dst = _stage_ppi_proj(
        num_dp, d_dp, dp_b, num_xp, d_xp, xp_b,
        ppi_W[:HID], ppi_W[HID:], ppi_as, ppi_ad)

    loops = jnp.arange(P_NUM, dtype=jnp.int32)
    ppi_src, ppi_seg = _pad_edges(
        jnp.concatenate([ppi_edge_index[0].astype(jnp.int32), loops]),
        jnp.concatenate([ppi_edge_index[1].astype(jnp.int32), loops]),
        e_ppi_pad, P_NUM)

    num_ppi, d_ppi = _edge_kernel(e_ppi_pad, P_PAD, P_PAD)(
        ppi_src, ppi_seg, s_src.reshape(-1), s_dst.reshape(-1), h_ppi)

    p_hid_pad, hb_pd, sb_pd, hb_px, sb_px = _stage_phid_proj(
        num_ppi, d_ppi, ppi_b, pd_Wb, pd_ab, px_Wb, px_ab)

    sa_pd = _matvec(drug_p, (pd_Wa @ pd_aa).reshape(HID, 1))
    sa_px = _matvec(dis_p, (px_Wa @ px_aa).reshape(HID, 1))

    pd_src, pd_seg = _pad_edges(drug_protein_edge_index[1],
                                drug_protein_edge_index[0], e_dp_pad, DR_NUM)
    px_src, px_seg = _pad_edges(disease_protein_edge_index[1],
                                disease_protein_edge_index[0], e_xp_pad,
                                DI_NUM)

    num_pd, d_pd = _edge_kernel(e_dp_pad, P_PAD, DR_PAD)(
        pd_src, pd_seg, sb_pd.reshape(-1), sa_pd.reshape(-1), hb_pd)
    sb_px_t, _ = lax.optimization_barrier((sb_px.reshape(-1), d_pd))
    num_px, d_px = _edge_kernel(e_xp_pad, P_PAD, DI_PAD)(
        px_src, px_seg, sb_px_t, sa_px.reshape(-1), hb_px)

    drug_out = _epi(num_pd, d_pd, pd_b)[:DR_NUM]
    disease_out = _epi(num_px, d_px, px_b)[:DI_NUM]
    return (drug_out, disease_out, p_hid_pad[:P_NUM])
